# baseline (speedup 1.0000x reference)
import functools

import jax
import jax.numpy as jnp
from jax import lax
from jax.experimental import pallas as pl
from jax.experimental.pallas import tpu as pltpu
from jax.experimental.pallas import tpu_sc as plsc

S = 2048
D = 1024
F = 2048
E = 8
T = 256
TD = 512
NT0 = S // TD
NTR = 15
SR = NTR * T

_SC_CORES = 2
_SC_SUBCORES = 16
_SC_WORKERS = _SC_CORES * _SC_SUBCORES
_ROWS_PER_W = S // _SC_WORKERS


def _router_body(x_ref, wr_ref, g0_ref, go_ref, dst_ref, meta_ref):
    x = x_ref[...]
    wr = wr_ref[...]
    logits = jnp.dot(x, wr, preferred_element_type=jnp.float32)
    m = jnp.max(logits, axis=1, keepdims=True)
    ex = jnp.exp(logits - m)
    p = ex / jnp.sum(ex, axis=1, keepdims=True)

    lane = lax.broadcasted_iota(jnp.int32, (S, E), 1)
    g0 = p[:, 0:1]
    pm = jnp.where(lane == 0, -1.0, p)
    go = jnp.max(pm, axis=1, keepdims=True)
    eo = jnp.min(jnp.where(pm == go, lane, E), axis=1, keepdims=True)

    ssum = jnp.clip(g0 + go, 1e-9, None)
    g0_ref[...] = g0 / ssum
    go_ref[...] = go / ssum

    oh = (lane == eo).astype(jnp.bfloat16)
    ri = lax.broadcasted_iota(jnp.int32, (S, S), 0)
    ci = lax.broadcasted_iota(jnp.int32, (S, S), 1)
    lt = (ri >= ci).astype(jnp.bfloat16)
    cum = jnp.dot(lt, oh, preferred_element_type=jnp.float32)
    rank = jnp.sum(jnp.where(lane == eo, cum, 0.0), axis=1, keepdims=True) - 1.0

    ones_col = jnp.ones((S, 1), jnp.bfloat16)
    counts = lax.dot_general(oh, ones_col, (((0,), (0,)), ((), ())),
                             preferred_element_type=jnp.float32)
    pcc = jnp.ceil(counts * (1.0 / T)) * T
    er = lax.broadcasted_iota(jnp.int32, (E, E), 0)
    ec = lax.broadcasted_iota(jnp.int32, (E, E), 1)
    strict_lt = (ec < er).astype(jnp.float32)
    offc = jnp.dot(strict_lt, pcc, preferred_element_type=jnp.float32)
    total = jnp.sum(pcc)

    offsel = jnp.dot(oh.astype(jnp.float32), offc,
                     preferred_element_type=jnp.float32)
    dst_ref[...] = (offsel + rank).astype(jnp.int32)

    kk = lax.broadcasted_iota(jnp.int32, (E, 128), 0)
    tt = lax.broadcasted_iota(jnp.int32, (E, 128), 1)
    tr = tt.astype(jnp.float32) * T
    started = jnp.where((offc <= tr) & (kk >= 1), 1, 0)
    te_row = jnp.sum(started, axis=0, keepdims=True)
    valid_row = jnp.where(tr[0:1, :] < total, 1, 0)
    rsel = lax.broadcasted_iota(jnp.int32, (8, 128), 0)
    meta_ref[...] = jnp.where(rsel == 0, jnp.broadcast_to(te_row, (8, 128)),
                              jnp.where(rsel == 1,
                                        jnp.broadcast_to(valid_row, (8, 128)),
                                        0))


def _router(x2d, wr):
    return pl.pallas_call(
        _router_body,
        grid=(1,),
        in_specs=[
            pl.BlockSpec((S, D), lambda i: (0, 0)),
            pl.BlockSpec((D, E), lambda i: (0, 0)),
        ],
        out_specs=[
            pl.BlockSpec((S, 1), lambda i: (0, 0)),
            pl.BlockSpec((S, 1), lambda i: (0, 0)),
            pl.BlockSpec((S, 1), lambda i: (0, 0)),
            pl.BlockSpec((8, 128), lambda i: (0, 0)),
        ],
        out_shape=[
            jax.ShapeDtypeStruct((S, 1), jnp.float32),
            jax.ShapeDtypeStruct((S, 1), jnp.float32),
            jax.ShapeDtypeStruct((S, 1), jnp.int32),
            jax.ShapeDtypeStruct((8, 128), jnp.int32),
        ],
    )(x2d, wr)


def _sc_mesh():
    return plsc.VectorSubcoreMesh(core_axis_name="c", subcore_axis_name="s")


def _sc_scatter(x2d, dst):
    @functools.partial(
        pl.kernel, mesh=_sc_mesh(),
        out_type=jax.ShapeDtypeStruct((SR, D), jnp.float32),
        scratch_types=[
            pltpu.VMEM((_ROWS_PER_W,), jnp.int32),
            pltpu.VMEM((_ROWS_PER_W, D), jnp.float32),
            pltpu.SemaphoreType.DMA,
        ],
    )
    def k(x_hbm, i_hbm, o_hbm, idx_v, rows_v, sem):
        wid = lax.axis_index("s") * _SC_CORES + lax.axis_index("c")
        base = wid * _ROWS_PER_W
        pltpu.sync_copy(i_hbm.at[pl.ds(base, _ROWS_PER_W)], idx_v)
        pltpu.sync_copy(x_hbm.at[pl.ds(base, _ROWS_PER_W)], rows_v)
        pltpu.async_copy(rows_v, o_hbm.at[idx_v], sem).wait()

    return k(x2d, dst)


def _sc_gather(ys, dst):
    @functools.partial(
        pl.kernel, mesh=_sc_mesh(),
        out_type=jax.ShapeDtypeStruct((S, D), jnp.float32),
        scratch_types=[
            pltpu.VMEM((_ROWS_PER_W,), jnp.int32),
            pltpu.VMEM((_ROWS_PER_W, D), jnp.float32),
            pltpu.SemaphoreType.DMA,
        ],
    )
    def k(y_hbm, i_hbm, o_hbm, idx_v, rows_v, sem):
        wid = lax.axis_index("s") * _SC_CORES + lax.axis_index("c")
        base = wid * _ROWS_PER_W
        pltpu.sync_copy(i_hbm.at[pl.ds(base, _ROWS_PER_W)], idx_v)
        pltpu.async_copy(y_hbm.at[idx_v], rows_v, sem).wait()
        pltpu.sync_copy(rows_v, o_hbm.at[pl.ds(base, _ROWS_PER_W)])

    return k(ys, dst)


def _make_fetch_wait_cast(w1_ref, w3_ref, w2_ref, stg1, stg3, stg2,
                          b1, b3, b2, sem1, sem3, sem2):
    def _fetch(e):
        pltpu.make_async_copy(w1_ref.at[e], stg1, sem1).start()
        pltpu.make_async_copy(w3_ref.at[e], stg3, sem3).start()
        pltpu.make_async_copy(w2_ref.at[e], stg2, sem2).start()

    def _wait():
        pltpu.make_async_copy(w1_ref.at[0], stg1, sem1).wait()
        pltpu.make_async_copy(w3_ref.at[0], stg3, sem3).wait()
        pltpu.make_async_copy(w2_ref.at[0], stg2, sem2).wait()

    def _cast():
        b1[...] = stg1[...].astype(jnp.bfloat16)
        b3[...] = stg3[...].astype(jnp.bfloat16)
        b2[...] = stg2[...].astype(jnp.bfloat16)

    return _fetch, _wait, _cast


def _swiglu_tile(src_ref, b1, b3, b2, ys_ref):
    xt = src_ref[...].astype(jnp.bfloat16)
    h1 = jnp.dot(xt, b1[...], preferred_element_type=jnp.float32)
    h3 = jnp.dot(xt, b3[...], preferred_element_type=jnp.float32)
    hb = (jax.nn.silu(h1) * h3).astype(jnp.bfloat16)
    ys_ref[...] = jnp.dot(hb, b2[...], preferred_element_type=jnp.float32)


_W_SCRATCH = [
    pltpu.VMEM((D, F), jnp.float32),
    pltpu.VMEM((D, F), jnp.float32),
    pltpu.VMEM((F, D), jnp.float32),
    pltpu.VMEM((D, F), jnp.bfloat16),
    pltpu.VMEM((D, F), jnp.bfloat16),
    pltpu.VMEM((F, D), jnp.bfloat16),
    pltpu.SMEM((1,), jnp.int32),
    pltpu.SemaphoreType.DMA,
    pltpu.SemaphoreType.DMA,
    pltpu.SemaphoreType.DMA,
]


def _dense_body(xd_ref, w1_ref, w3_ref, w2_ref, ys_ref,
                stg1, stg3, stg2, b1, b3, b2, pfe, sem1, sem3, sem2):
    t = pl.program_id(0)
    _fetch, _wait, _cast = _make_fetch_wait_cast(
        w1_ref, w3_ref, w2_ref, stg1, stg3, stg2, b1, b3, b2,
        sem1, sem3, sem2)

    @pl.when(t == 0)
    def _():
        _fetch(0)
        _wait()
        _cast()

    _swiglu_tile(xd_ref, b1, b3, b2, ys_ref)


def _dense(x2d, w1f, w3f, w2f):
    return pl.pallas_call(
        _dense_body,
        grid=(NT0,),
        in_specs=[
            pl.BlockSpec((TD, D), lambda t: (t, 0)),
            pl.BlockSpec(memory_space=pltpu.MemorySpace.HBM),
            pl.BlockSpec(memory_space=pltpu.MemorySpace.HBM),
            pl.BlockSpec(memory_space=pltpu.MemorySpace.HBM),
        ],
        out_specs=pl.BlockSpec((TD, D), lambda t: (t, 0)),
        out_shape=jax.ShapeDtypeStruct((S, D), jnp.float32),
        scratch_shapes=_W_SCRATCH,
    )(x2d, w1f, w3f, w2f)


def _gmm_body(te_ref, vd_ref, xs_ref, w1_ref, w3_ref, w2_ref, ys_ref,
              stg1, stg3, stg2, b1, b3, b2, pfe, sem1, sem3, sem2):
    t = pl.program_id(0)
    _fetch, _wait, _cast = _make_fetch_wait_cast(
        w1_ref, w3_ref, w2_ref, stg1, stg3, stg2, b1, b3, b2,
        sem1, sem3, sem2)

    @pl.when(t == 0)
    def _():
        e0 = te_ref[0]
        _fetch(e0)
        _wait()
        _cast()
        nxt = jnp.minimum(e0 + 1, E - 1)

        @pl.when(e0 < E - 1)
        def _():
            _fetch(nxt)

        pfe[0] = jnp.where(e0 < E - 1, nxt, -1)

    @pl.when((t > 0) & (te_ref[t] != te_ref[jnp.maximum(t - 1, 0)]))
    def _():
        _wait()

        @pl.when(pfe[0] != te_ref[t])
        def _():
            _fetch(te_ref[t])
            _wait()

        _cast()
        nxt = jnp.minimum(te_ref[t] + 1, E - 1)

        @pl.when(te_ref[t] < E - 1)
        def _():
            _fetch(nxt)

        pfe[0] = jnp.where(te_ref[t] < E - 1, nxt, -1)

    @pl.when(vd_ref[t] == 1)
    def _():
        _swiglu_tile(xs_ref, b1, b3, b2, ys_ref)

    @pl.when((t == NTR - 1) & (pfe[0] >= 0))
    def _():
        _wait()
        pfe[0] = -1


def _gmm(xs, w1f, w3f, w2f, te, vd):
    grid_spec = pltpu.PrefetchScalarGridSpec(
        num_scalar_prefetch=2,
        grid=(NTR,),
        in_specs=[
            pl.BlockSpec((T, D), lambda t, te, vd: (t, 0)),
            pl.BlockSpec(memory_space=pltpu.MemorySpace.HBM),
            pl.BlockSpec(memory_space=pltpu.MemorySpace.HBM),
            pl.BlockSpec(memory_space=pltpu.MemorySpace.HBM),
        ],
        out_specs=pl.BlockSpec((T, D), lambda t, te, vd: (t, 0)),
        scratch_shapes=_W_SCRATCH,
    )
    return pl.pallas_call(
        _gmm_body,
        grid_spec=grid_spec,
        out_shape=jax.ShapeDtypeStruct((SR, D), jnp.float32),
    )(te, vd, xs, w1f, w3f, w2f)


def _combine_body(y0_ref, g_ref, g0_ref, go_ref, out_ref):
    out_ref[...] = y0_ref[...] * g0_ref[...] + g_ref[...] * go_ref[...]


def _combine(ys, g, g0, go):
    return pl.pallas_call(
        _combine_body,
        grid=(S // T,),
        in_specs=[
            pl.BlockSpec((T, D), lambda t: (t, 0)),
            pl.BlockSpec((T, D), lambda t: (t, 0)),
            pl.BlockSpec((T, 1), lambda t: (t, 0)),
            pl.BlockSpec((T, 1), lambda t: (t, 0)),
        ],
        out_specs=pl.BlockSpec((T, D), lambda t: (t, 0)),
        out_shape=jax.ShapeDtypeStruct((S, D), jnp.float32),
    )(ys, g, g0, go)


def kernel(x, Wr, W1, W3, W2):
    B, Sx, Dx = x.shape
    x2d = x.reshape(S, D)

    g0, go, dst2d, meta = _router(x2d, Wr)
    dst = dst2d.reshape(S)
    te = meta[0, :NTR]
    vd = meta[1, :NTR]

    xs = _sc_scatter(x2d, dst)
    y0 = _dense(x2d, W1, W3, W2)
    ys = _gmm(xs, W1, W3, W2, te, vd)
    g = _sc_gather(ys, dst)
    out = _combine(y0, g, g0, go)
    return out.reshape(B, Sx, Dx)


# speedup vs baseline: 1.0405x; 1.0405x over previous
import functools

import jax
import jax.numpy as jnp
from jax import lax
from jax.experimental import pallas as pl
from jax.experimental.pallas import tpu as pltpu
from jax.experimental.pallas import tpu_sc as plsc

S = 2048
D = 1024
F = 2048
E = 8
T = 256
NT0 = S // T
NTR = 15
NTT = NT0 + NTR
SR = NTT * T

_SC_CORES = 2
_SC_SUBCORES = 16
_SC_WORKERS = _SC_CORES * _SC_SUBCORES
_ROWS_PER_W = S // _SC_WORKERS


def _router_body(x_ref, wr_ref, g0_ref, go_ref, dst_ref, meta_ref):
    x = x_ref[...]
    wr = wr_ref[...]
    logits = jnp.dot(x, wr, preferred_element_type=jnp.float32)
    m = jnp.max(logits, axis=1, keepdims=True)
    ex = jnp.exp(logits - m)
    p = ex / jnp.sum(ex, axis=1, keepdims=True)

    lane = lax.broadcasted_iota(jnp.int32, (S, E), 1)
    g0 = p[:, 0:1]
    pm = jnp.where(lane == 0, -1.0, p)
    go = jnp.max(pm, axis=1, keepdims=True)
    eo = jnp.min(jnp.where(pm == go, lane, E), axis=1, keepdims=True)

    ssum = jnp.clip(g0 + go, 1e-9, None)
    g0_ref[...] = g0 / ssum
    go_ref[...] = go / ssum

    oh = (lane == eo).astype(jnp.bfloat16)
    ri = lax.broadcasted_iota(jnp.int32, (S, S), 0)
    ci = lax.broadcasted_iota(jnp.int32, (S, S), 1)
    lt = (ri >= ci).astype(jnp.bfloat16)
    cum = jnp.dot(lt, oh, preferred_element_type=jnp.float32)
    rank = jnp.sum(jnp.where(lane == eo, cum, 0.0), axis=1, keepdims=True) - 1.0

    ones_col = jnp.ones((S, 1), jnp.bfloat16)
    counts = lax.dot_general(oh, ones_col, (((0,), (0,)), ((), ())),
                             preferred_element_type=jnp.float32)
    pcc = jnp.ceil(counts * (1.0 / T)) * T
    er = lax.broadcasted_iota(jnp.int32, (E, E), 0)
    ec = lax.broadcasted_iota(jnp.int32, (E, E), 1)
    strict_lt = (ec < er).astype(jnp.float32)
    offc = jnp.dot(strict_lt, pcc, preferred_element_type=jnp.float32)
    total = jnp.sum(pcc)

    offsel = jnp.dot(oh.astype(jnp.float32), offc,
                     preferred_element_type=jnp.float32)
    dst_ref[...] = (offsel + rank).astype(jnp.int32) + S

    kk = lax.broadcasted_iota(jnp.int32, (E, 128), 0)
    tt = lax.broadcasted_iota(jnp.int32, (E, 128), 1)
    tr = (tt - NT0).astype(jnp.float32) * T
    started = jnp.where((offc <= tr) & (kk >= 1) & (tt >= NT0), 1, 0)
    te_row = jnp.sum(started, axis=0, keepdims=True)
    valid_row = jnp.where((tt[0:1, :] < NT0) | (tr[0:1, :] < total), 1, 0)
    rsel = lax.broadcasted_iota(jnp.int32, (8, 128), 0)
    meta_ref[...] = jnp.where(rsel == 0, jnp.broadcast_to(te_row, (8, 128)),
                              jnp.where(rsel == 1,
                                        jnp.broadcast_to(valid_row, (8, 128)),
                                        0))


def _router(x2d, wr):
    return pl.pallas_call(
        _router_body,
        grid=(1,),
        in_specs=[
            pl.BlockSpec((S, D), lambda i: (0, 0)),
            pl.BlockSpec((D, E), lambda i: (0, 0)),
        ],
        out_specs=[
            pl.BlockSpec((S, 1), lambda i: (0, 0)),
            pl.BlockSpec((S, 1), lambda i: (0, 0)),
            pl.BlockSpec((S, 1), lambda i: (0, 0)),
            pl.BlockSpec((8, 128), lambda i: (0, 0)),
        ],
        out_shape=[
            jax.ShapeDtypeStruct((S, 1), jnp.float32),
            jax.ShapeDtypeStruct((S, 1), jnp.float32),
            jax.ShapeDtypeStruct((S, 1), jnp.int32),
            jax.ShapeDtypeStruct((8, 128), jnp.int32),
        ],
    )(x2d, wr)


def _sc_mesh():
    return plsc.VectorSubcoreMesh(core_axis_name="c", subcore_axis_name="s")


def _sc_scatter(x2d, dst):
    @functools.partial(
        pl.kernel, mesh=_sc_mesh(),
        out_type=jax.ShapeDtypeStruct((SR, D), jnp.float32),
        scratch_types=[
            pltpu.VMEM((_ROWS_PER_W,), jnp.int32),
            pltpu.VMEM((_ROWS_PER_W, D), jnp.float32),
            pltpu.SemaphoreType.DMA,
        ],
    )
    def k(x_hbm, i_hbm, o_hbm, idx_v, rows_v, sem):
        wid = lax.axis_index("s") * _SC_CORES + lax.axis_index("c")
        base = wid * _ROWS_PER_W
        pltpu.sync_copy(i_hbm.at[pl.ds(base, _ROWS_PER_W)], idx_v)
        pltpu.sync_copy(x_hbm.at[pl.ds(base, _ROWS_PER_W)], rows_v)
        pltpu.async_copy(rows_v, o_hbm.at[idx_v], sem).wait()

    return k(x2d, dst)


def _sc_gather(ys, dst):
    @functools.partial(
        pl.kernel, mesh=_sc_mesh(),
        out_type=jax.ShapeDtypeStruct((S, D), jnp.float32),
        scratch_types=[
            pltpu.VMEM((_ROWS_PER_W,), jnp.int32),
            pltpu.VMEM((_ROWS_PER_W, D), jnp.float32),
            pltpu.SemaphoreType.DMA,
        ],
    )
    def k(y_hbm, i_hbm, o_hbm, idx_v, rows_v, sem):
        wid = lax.axis_index("s") * _SC_CORES + lax.axis_index("c")
        base = wid * _ROWS_PER_W
        pltpu.sync_copy(i_hbm.at[pl.ds(base, _ROWS_PER_W)], idx_v)
        pltpu.async_copy(y_hbm.at[idx_v], rows_v, sem).wait()
        pltpu.sync_copy(rows_v, o_hbm.at[pl.ds(base, _ROWS_PER_W)])

    return k(ys, dst)


def _make_fetch_wait_cast(w1_ref, w3_ref, w2_ref, stg1, stg3, stg2,
                          b1, b3, b2, sem1, sem3, sem2):
    def _fetch(e):
        pltpu.make_async_copy(w1_ref.at[e], stg1, sem1).start()
        pltpu.make_async_copy(w3_ref.at[e], stg3, sem3).start()
        pltpu.make_async_copy(w2_ref.at[e], stg2, sem2).start()

    def _wait():
        pltpu.make_async_copy(w1_ref.at[0], stg1, sem1).wait()
        pltpu.make_async_copy(w3_ref.at[0], stg3, sem3).wait()
        pltpu.make_async_copy(w2_ref.at[0], stg2, sem2).wait()

    def _cast():
        b1[...] = stg1[...].astype(jnp.bfloat16)
        b3[...] = stg3[...].astype(jnp.bfloat16)
        b2[...] = stg2[...].astype(jnp.bfloat16)

    return _fetch, _wait, _cast


def _swiglu_tile(src_ref, b1, b3, b2, ys_ref):
    xt = src_ref[...].astype(jnp.bfloat16)
    h1 = jnp.dot(xt, b1[...], preferred_element_type=jnp.float32)
    h3 = jnp.dot(xt, b3[...], preferred_element_type=jnp.float32)
    hb = (jax.nn.silu(h1) * h3).astype(jnp.bfloat16)
    ys_ref[...] = jnp.dot(hb, b2[...], preferred_element_type=jnp.float32)


_W_SCRATCH = [
    pltpu.VMEM((D, F), jnp.float32),
    pltpu.VMEM((D, F), jnp.float32),
    pltpu.VMEM((F, D), jnp.float32),
    pltpu.VMEM((D, F), jnp.bfloat16),
    pltpu.VMEM((D, F), jnp.bfloat16),
    pltpu.VMEM((F, D), jnp.bfloat16),
    pltpu.SMEM((1,), jnp.int32),
    pltpu.SemaphoreType.DMA,
    pltpu.SemaphoreType.DMA,
    pltpu.SemaphoreType.DMA,
]


def _gmm_body(te_ref, vd_ref, xd_ref, xs_ref, w1_ref, w3_ref, w2_ref, ys_ref,
              stg1, stg3, stg2, b1, b3, b2, pfe, sem1, sem3, sem2):
    t = pl.program_id(0)
    _fetch, _wait, _cast = _make_fetch_wait_cast(
        w1_ref, w3_ref, w2_ref, stg1, stg3, stg2, b1, b3, b2,
        sem1, sem3, sem2)

    @pl.when(t == 0)
    def _():
        _fetch(0)
        _wait()
        _cast()
        _fetch(1)
        pfe[0] = 1

    @pl.when((t > 0) & (te_ref[t] != te_ref[jnp.maximum(t - 1, 0)]))
    def _():
        _wait()

        @pl.when(pfe[0] != te_ref[t])
        def _():
            _fetch(te_ref[t])
            _wait()

        _cast()
        nxt = jnp.minimum(te_ref[t] + 1, E - 1)

        @pl.when(te_ref[t] < E - 1)
        def _():
            _fetch(nxt)

        pfe[0] = jnp.where(te_ref[t] < E - 1, nxt, -1)

    @pl.when(t < NT0)
    def _():
        _swiglu_tile(xd_ref, b1, b3, b2, ys_ref)

    @pl.when((t >= NT0) & (vd_ref[t] == 1))
    def _():
        _swiglu_tile(xs_ref, b1, b3, b2, ys_ref)

    @pl.when((t == NTT - 1) & (pfe[0] >= 0))
    def _():
        _wait()
        pfe[0] = -1


def _gmm(x2d, xs, w1f, w3f, w2f, te, vd):
    grid_spec = pltpu.PrefetchScalarGridSpec(
        num_scalar_prefetch=2,
        grid=(NTT,),
        in_specs=[
            pl.BlockSpec((T, D), lambda t, te, vd: (jnp.minimum(t, NT0 - 1), 0)),
            pl.BlockSpec((T, D), lambda t, te, vd: (jnp.maximum(t, NT0), 0)),
            pl.BlockSpec(memory_space=pltpu.MemorySpace.HBM),
            pl.BlockSpec(memory_space=pltpu.MemorySpace.HBM),
            pl.BlockSpec(memory_space=pltpu.MemorySpace.HBM),
        ],
        out_specs=pl.BlockSpec((T, D), lambda t, te, vd: (t, 0)),
        scratch_shapes=_W_SCRATCH,
    )
    return pl.pallas_call(
        _gmm_body,
        grid_spec=grid_spec,
        out_shape=jax.ShapeDtypeStruct((SR, D), jnp.float32),
    )(te, vd, x2d, xs, w1f, w3f, w2f)


def _combine_body(y0_ref, g_ref, g0_ref, go_ref, out_ref):
    out_ref[...] = y0_ref[...] * g0_ref[...] + g_ref[...] * go_ref[...]


def _combine(ys, g, g0, go):
    return pl.pallas_call(
        _combine_body,
        grid=(S // T,),
        in_specs=[
            pl.BlockSpec((T, D), lambda t: (t, 0)),
            pl.BlockSpec((T, D), lambda t: (t, 0)),
            pl.BlockSpec((T, 1), lambda t: (t, 0)),
            pl.BlockSpec((T, 1), lambda t: (t, 0)),
        ],
        out_specs=pl.BlockSpec((T, D), lambda t: (t, 0)),
        out_shape=jax.ShapeDtypeStruct((S, D), jnp.float32),
    )(ys, g, g0, go)


def kernel(x, Wr, W1, W3, W2):
    B, Sx, Dx = x.shape
    x2d = x.reshape(S, D)

    g0, go, dst2d, meta = _router(x2d, Wr)
    dst = dst2d.reshape(S)
    te = meta[0, :NTT]
    vd = meta[1, :NTT]

    xs = _sc_scatter(x2d, dst)
    ys = _gmm(x2d, xs, W1, W3, W2, te, vd)
    g = _sc_gather(ys, dst)
    out = _combine(ys, g, g0, go)
    return out.reshape(B, Sx, Dx)
